# Initial kernel scaffold; baseline (speedup 1.0000x reference)
import jax
import jax.numpy as jnp
from jax.experimental import pallas as pl


def kernel(x, edge_index, edge_weight, batch, eps1, W1, b1, eps2, W2, b2, eps3, W3, b3, g1, bb1, g2, bb2, g3, bb3, fcW1, fcb1, fcW2, fcb2):
    raise NotImplementedError("write your pallas kernel here")


# speedup vs baseline: 4.1364x; 4.1364x over previous
import functools

import jax
import jax.numpy as jnp
from jax import lax
from jax.experimental import pallas as pl
from jax.experimental.pallas import tpu as pltpu
from jax.experimental.pallas import tpu_sc as plsc

_NC = 2
_NS = 16
_L = 16


def _sc_agg(x, src, dst, ew):
    n, d = x.shape
    e = src.shape[0]
    nw = _NC * _NS
    epw = e // nw
    K = 80
    nch = epw // K
    rpt = (n // (8 * _NS)) * 8
    rem = n - _NS * rpt
    ZR = 156
    assert e == nw * epw and epw == nch * K and rpt % ZR == 0
    assert d % _L == 0 and K % 8 == 0 and epw % 8 == 0
    assert rem % 8 == 0 and rem <= ZR

    mesh = plsc.VectorSubcoreMesh(core_axis_name="c", subcore_axis_name="s")

    @functools.partial(
        pl.kernel,
        out_type=jax.ShapeDtypeStruct((_NC, n, d), jnp.float32),
        mesh=mesh,
        scratch_types=[
            pltpu.VMEM((K,), jnp.int32),
            pltpu.VMEM((K,), jnp.int32),
            pltpu.VMEM((K,), jnp.float32),
            pltpu.VMEM((K, d), jnp.float32),
            pltpu.VMEM((ZR, d), jnp.float32),
            pltpu.VMEM_SHARED((n, d), jnp.float32),
            pltpu.SemaphoreType.DMA,
        ],
    )
    def agg_kernel(x_hbm, src_hbm, dst_hbm, ew_hbm, out_hbm,
                   src_v, dst_v, ew_v, rows_v, zbuf_v, acc_sh, sem):
        c = lax.axis_index("c")
        s = lax.axis_index("s")
        wid = c * _NS + s
        row0 = s * rpt

        z16 = jnp.zeros((_L,), jnp.float32)

        def zrow(r, carry):
            for j in range(d // _L):
                zbuf_v[r, pl.ds(j * _L, _L)] = z16
            return carry

        lax.fori_loop(0, ZR, zrow, 0)
        for t in range(rpt // ZR):
            pltpu.sync_copy(zbuf_v, acc_sh.at[pl.ds(row0 + t * ZR, ZR)])
        if rem:
            @pl.when(s == _NS - 1)
            def _():
                pltpu.sync_copy(zbuf_v.at[pl.ds(0, rem)],
                                acc_sh.at[pl.ds(n - rem, rem)])
        plsc.subcore_barrier()

        ebase = wid * epw

        def chunk(i, carry):
            base = ebase + i * K
            pltpu.sync_copy(src_hbm.at[pl.ds(base, K)], src_v)
            pltpu.sync_copy(dst_hbm.at[pl.ds(base, K)], dst_v)
            pltpu.sync_copy(ew_hbm.at[pl.ds(base, K)], ew_v)
            pltpu.async_copy(x_hbm.at[src_v], rows_v, sem).wait()

            def scale(q, cc):
                ewv = ew_v[pl.ds(q * _L, _L)]
                for l in range(_L):
                    w = lax.gather(
                        ewv, jnp.full((_L, 1), l, jnp.int32),
                        lax.GatherDimensionNumbers(
                            offset_dims=(), collapsed_slice_dims=(0,),
                            start_index_map=(0,)),
                        (1,), mode=lax.GatherScatterMode.PROMISE_IN_BOUNDS)
                    k = q * _L + l
                    for j in range(d // _L):
                        rows_v[k, pl.ds(j * _L, _L)] = (
                            rows_v[k, pl.ds(j * _L, _L)] * w)
                return cc

            lax.fori_loop(0, K // _L, scale, 0)
            pltpu.sync_copy(rows_v, acc_sh.at[dst_v], add=True)
            return carry

        lax.fori_loop(0, nch, chunk, 0)
        plsc.subcore_barrier()
        pltpu.sync_copy(acc_sh.at[pl.ds(row0, rpt)],
                        out_hbm.at[c].at[pl.ds(row0, rpt)])
        if rem:
            @pl.when(s == _NS - 1)
            def _():
                pltpu.sync_copy(acc_sh.at[pl.ds(n - rem, rem)],
                                out_hbm.at[c].at[pl.ds(n - rem, rem)])

    return agg_kernel(x, src, dst, ew)


def _layer_body(relu, h_ref, a_ref, eps_ref, w_ref, b_ref, g_ref, bb_ref, o_ref):
    agg = a_ref[0] + a_ref[1]
    eps = eps_ref[0]
    h = (1.0 + eps) * h_ref[...] + agg
    y = jnp.dot(h, w_ref[...], preferred_element_type=jnp.float32,
                precision=lax.Precision.HIGHEST) + b_ref[...]
    if relu:
        y = jnp.maximum(y, 0.0)
    m = jnp.mean(y, axis=0, keepdims=True)
    v = jnp.mean((y - m) * (y - m), axis=0, keepdims=True)
    o_ref[...] = (y - m) * lax.rsqrt(v + 1e-5) * g_ref[...] + bb_ref[...]


def _tc_layer(h, aggp, eps, w, b, g, bb, relu):
    n, dd = h.shape
    hh = w.shape[1]
    return pl.pallas_call(
        functools.partial(_layer_body, relu),
        out_shape=jax.ShapeDtypeStruct((n, hh), jnp.float32),
        in_specs=[
            pl.BlockSpec(memory_space=pltpu.VMEM),
            pl.BlockSpec(memory_space=pltpu.VMEM),
            pl.BlockSpec(memory_space=pltpu.SMEM),
            pl.BlockSpec(memory_space=pltpu.VMEM),
            pl.BlockSpec(memory_space=pltpu.VMEM),
            pl.BlockSpec(memory_space=pltpu.VMEM),
            pl.BlockSpec(memory_space=pltpu.VMEM),
        ],
        out_specs=pl.BlockSpec(memory_space=pltpu.VMEM),
    )(h, aggp, eps.reshape(1), w, b.reshape(1, -1), g.reshape(1, -1),
      bb.reshape(1, -1))


def _head_body(g_count, h_ref, batch_ref, w1_ref, b1_ref, w2_ref, b2_ref, o_ref):
    h = h_ref[...]
    n = h.shape[0]
    bt = batch_ref[...]
    gid = lax.broadcasted_iota(jnp.int32, (g_count, n), 0)
    oh = (gid == bt).astype(jnp.float32)
    pooled = jnp.dot(oh, h, preferred_element_type=jnp.float32,
                     precision=lax.Precision.HIGHEST)
    pooled = jnp.maximum(pooled, 0.0)
    y = jnp.maximum(jnp.dot(pooled, w1_ref[...], preferred_element_type=jnp.float32,
                            precision=lax.Precision.HIGHEST) + b1_ref[...], 0.0)
    o_ref[...] = jnp.dot(y, w2_ref[...], preferred_element_type=jnp.float32,
                         precision=lax.Precision.HIGHEST) + b2_ref[...]


def _tc_head(h, batch, g_count, w1, b1, w2, b2):
    return pl.pallas_call(
        functools.partial(_head_body, g_count),
        out_shape=jax.ShapeDtypeStruct((g_count, 1), jnp.float32),
    )(h, batch.reshape(1, -1), w1, b1.reshape(1, -1), w2, b2.reshape(1, -1))


def kernel(x, edge_index, edge_weight, batch, eps1, W1, b1, eps2, W2, b2,
           eps3, W3, b3, g1, bb1, g2, bb2, g3, bb3, fcW1, fcb1, fcW2, fcb2):
    src = edge_index[0]
    dst = edge_index[1]
    g_count = 64

    aggp = _sc_agg(x, src, dst, edge_weight)
    h = _tc_layer(x, aggp, eps1, W1, b1, g1, bb1, relu=True)
    aggp = _sc_agg(h, src, dst, edge_weight)
    h = _tc_layer(h, aggp, eps2, W2, b2, g2, bb2, relu=True)
    aggp = _sc_agg(h, src, dst, edge_weight)
    h = _tc_layer(h, aggp, eps3, W3, b3, g3, bb3, relu=False)
    out = _tc_head(h, batch, g_count, fcW1, fcb1, fcW2, fcb2)
    return out.reshape(-1)


# speedup vs baseline: 9.9544x; 2.4066x over previous
import functools

import jax
import jax.numpy as jnp
from jax import lax
from jax.experimental import pallas as pl
from jax.experimental.pallas import tpu as pltpu
from jax.experimental.pallas import tpu_sc as plsc

_NC = 2
_NS = 16
_L = 16


def _sc_agg(x, packed, ew):
    n, d = x.shape
    e = packed.shape[0]
    nw = _NC * _NS
    epw = e // nw
    K = 40
    nch = epw // K
    rpt = (n // (8 * _NS)) * 8
    rem = n - _NS * rpt
    assert e == nw * epw and epw == nch * K and nch % 2 == 0
    assert d % _L == 0 and K % 8 == 0 and epw % 8 == 0
    assert rem % 8 == 0 and rem <= K

    segs = []
    off = 0
    while off + _L <= K:
        segs.append((off, 0))
        off += _L
    if off < K:
        segs.append((K - _L, _L - (K - off)))

    mesh = plsc.VectorSubcoreMesh(core_axis_name="c", subcore_axis_name="s")

    @functools.partial(
        pl.kernel,
        out_type=jax.ShapeDtypeStruct((_NC, n, d), jnp.float32),
        mesh=mesh,
        scratch_types=[
            pltpu.VMEM((epw,), jnp.int32),
            pltpu.VMEM((epw,), jnp.float32),
            pltpu.VMEM((K,), jnp.int32),
            pltpu.VMEM((K,), jnp.int32),
            pltpu.VMEM((2, K), jnp.int32),
            pltpu.VMEM((2, K), jnp.int32),
            pltpu.VMEM((K, d), jnp.float32),
            pltpu.VMEM((K, d), jnp.float32),
            pltpu.VMEM((K, d), jnp.float32),
            pltpu.VMEM((K, d), jnp.float32),
            pltpu.VMEM_SHARED((n, d), jnp.float32),
            pltpu.SemaphoreType.DMA,
            pltpu.SemaphoreType.DMA,
            pltpu.SemaphoreType.DMA,
            pltpu.SemaphoreType.DMA,
        ],
    )
    def agg_kernel(x_hbm, packed_hbm, ew_hbm, out_hbm,
                   packed_v, ew_v, srcb_a, srcb_b, dstb_a, dstb_b,
                   ga_v, gb_v, sa_v, sb_v, acc_sh,
                   sem_ga, sem_gb, sem_sa, sem_sb):
        c = lax.axis_index("c")
        s = lax.axis_index("s")
        wid = c * _NS + s
        row0 = s * rpt
        ebase = wid * epw

        pltpu.sync_copy(packed_hbm.at[pl.ds(ebase, epw)], packed_v)
        pltpu.sync_copy(ew_hbm.at[pl.ds(ebase, epw)], ew_v)

        def unpack(i, srcb, dstb, slot):
            for soff, _l0 in segs:
                p = packed_v[pl.ds(i * K + soff, _L)]
                srcb[pl.ds(soff, _L)] = p & jnp.int32(0xFFFF)
                dstb[slot, pl.ds(soff, _L)] = lax.shift_right_logical(
                    p, jnp.int32(16))

        def gather(srcb, g_v, sem):
            pltpu.async_copy(x_hbm.at[srcb], g_v, sem)

        def wait_gather(srcb, g_v, sem):
            pltpu.make_async_copy(x_hbm.at[srcb], g_v, sem).wait()

        def scatter(dstb, slot, s_v, sem):
            pltpu.async_copy(s_v, acc_sh.at[dstb.at[slot]], sem, add=True)

        def wait_scatter(dstb, slot, s_v, sem):
            pltpu.make_async_copy(s_v, acc_sh.at[dstb.at[slot]], sem).wait()

        unpack(0, srcb_a, dstb_a, 0)
        unpack(1, srcb_b, dstb_b, 0)
        gather(srcb_a, ga_v, sem_ga)
        gather(srcb_b, gb_v, sem_gb)

        z16 = jnp.zeros((_L,), jnp.float32)
        for k in range(K):
            for j in range(d // _L):
                sa_v[k, pl.ds(j * _L, _L)] = z16
        nfull = rpt // K
        tail = rpt - nfull * K
        for t in range(nfull):
            pltpu.sync_copy(sa_v, acc_sh.at[pl.ds(row0 + t * K, K)])
        if tail:
            pltpu.sync_copy(sa_v.at[pl.ds(0, tail)],
                            acc_sh.at[pl.ds(row0 + nfull * K, tail)])
        if rem:
            @pl.when(s == _NS - 1)
            def _():
                pltpu.sync_copy(sa_v.at[pl.ds(0, rem)],
                                acc_sh.at[pl.ds(n - rem, rem)])
        plsc.subcore_barrier()

        def scale(g_v, s_v, i):
            for soff, l0 in segs:
                ewv = ew_v[pl.ds(i * K + soff, _L)]
                for l in range(l0, _L):
                    w = lax.gather(
                        ewv, jnp.full((_L, 1), l, jnp.int32),
                        lax.GatherDimensionNumbers(
                            offset_dims=(), collapsed_slice_dims=(0,),
                            start_index_map=(0,)),
                        (1,), mode=lax.GatherScatterMode.PROMISE_IN_BOUNDS)
                    k = soff + l
                    for j in range(d // _L):
                        s_v[k, pl.ds(j * _L, _L)] = (
                            g_v[k, pl.ds(j * _L, _L)] * w)

        def step(i, slot_cur, slot_next, srcb, dstb, g_v, s_v, sem_g, sem_s):
            wait_gather(srcb, g_v, sem_g)

            @pl.when(i >= 2)
            def _():
                wait_scatter(dstb, slot_next, s_v, sem_s)

            scale(g_v, s_v, i)
            scatter(dstb, slot_cur, s_v, sem_s)

            @pl.when(i + 2 < nch)
            def _():
                unpack(i + 2, srcb, dstb, slot_next)
                gather(srcb, g_v, sem_g)

        def pair(p, carry):
            slot_cur = lax.rem(p, 2)
            slot_next = 1 - slot_cur
            step(2 * p, slot_cur, slot_next, srcb_a, dstb_a, ga_v, sa_v,
                 sem_ga, sem_sa)
            step(2 * p + 1, slot_cur, slot_next, srcb_b, dstb_b, gb_v, sb_v,
                 sem_gb, sem_sb)
            return carry

        lax.fori_loop(0, nch // 2, pair, 0)
        last_slot = (nch // 2 - 1) % 2
        wait_scatter(dstb_a, last_slot, sa_v, sem_sa)
        wait_scatter(dstb_b, last_slot, sb_v, sem_sb)

        plsc.subcore_barrier()
        pltpu.sync_copy(acc_sh.at[pl.ds(row0, rpt)],
                        out_hbm.at[c].at[pl.ds(row0, rpt)])
        if rem:
            @pl.when(s == _NS - 1)
            def _():
                pltpu.sync_copy(acc_sh.at[pl.ds(n - rem, rem)],
                                out_hbm.at[c].at[pl.ds(n - rem, rem)])

    return agg_kernel(x, packed, ew)


def _layer_body(relu, h_ref, a_ref, eps_ref, w_ref, b_ref, g_ref, bb_ref, o_ref):
    agg = a_ref[0] + a_ref[1]
    eps = eps_ref[0]
    h = (1.0 + eps) * h_ref[...] + agg
    y = jnp.dot(h, w_ref[...], preferred_element_type=jnp.float32,
                precision=lax.Precision.HIGHEST) + b_ref[...]
    if relu:
        y = jnp.maximum(y, 0.0)
    m = jnp.mean(y, axis=0, keepdims=True)
    v = jnp.mean((y - m) * (y - m), axis=0, keepdims=True)
    o_ref[...] = (y - m) * lax.rsqrt(v + 1e-5) * g_ref[...] + bb_ref[...]


def _tc_layer(h, aggp, eps, w, b, g, bb, relu):
    n, dd = h.shape
    hh = w.shape[1]
    return pl.pallas_call(
        functools.partial(_layer_body, relu),
        out_shape=jax.ShapeDtypeStruct((n, hh), jnp.float32),
        in_specs=[
            pl.BlockSpec(memory_space=pltpu.VMEM),
            pl.BlockSpec(memory_space=pltpu.VMEM),
            pl.BlockSpec(memory_space=pltpu.SMEM),
            pl.BlockSpec(memory_space=pltpu.VMEM),
            pl.BlockSpec(memory_space=pltpu.VMEM),
            pl.BlockSpec(memory_space=pltpu.VMEM),
            pl.BlockSpec(memory_space=pltpu.VMEM),
        ],
        out_specs=pl.BlockSpec(memory_space=pltpu.VMEM),
    )(h, aggp, eps.reshape(1), w, b.reshape(1, -1), g.reshape(1, -1),
      bb.reshape(1, -1))


def _head_body(g_count, h_ref, batch_ref, w1_ref, b1_ref, w2_ref, b2_ref, o_ref):
    h = h_ref[...]
    n = h.shape[0]
    bt = batch_ref[...]
    gid = lax.broadcasted_iota(jnp.int32, (g_count, n), 0)
    oh = (gid == bt).astype(jnp.float32)
    pooled = jnp.dot(oh, h, preferred_element_type=jnp.float32,
                     precision=lax.Precision.HIGHEST)
    pooled = jnp.maximum(pooled, 0.0)
    y = jnp.maximum(jnp.dot(pooled, w1_ref[...], preferred_element_type=jnp.float32,
                            precision=lax.Precision.HIGHEST) + b1_ref[...], 0.0)
    o_ref[...] = jnp.dot(y, w2_ref[...], preferred_element_type=jnp.float32,
                         precision=lax.Precision.HIGHEST) + b2_ref[...]


def _tc_head(h, batch, g_count, w1, b1, w2, b2):
    return pl.pallas_call(
        functools.partial(_head_body, g_count),
        out_shape=jax.ShapeDtypeStruct((g_count, 1), jnp.float32),
    )(h, batch.reshape(1, -1), w1, b1.reshape(1, -1), w2, b2.reshape(1, -1))


def kernel(x, edge_index, edge_weight, batch, eps1, W1, b1, eps2, W2, b2,
           eps3, W3, b3, g1, bb1, g2, bb2, g3, bb3, fcW1, fcb1, fcW2, fcb2):
    src = edge_index[0]
    dst = edge_index[1]
    packed = jnp.bitwise_or(lax.shift_left(dst, 16), src)
    g_count = 64

    aggp = _sc_agg(x, packed, edge_weight)
    h = _tc_layer(x, aggp, eps1, W1, b1, g1, bb1, relu=True)
    aggp = _sc_agg(h, packed, edge_weight)
    h = _tc_layer(h, aggp, eps2, W2, b2, g2, bb2, relu=True)
    aggp = _sc_agg(h, packed, edge_weight)
    h = _tc_layer(h, aggp, eps3, W3, b3, g3, bb3, relu=False)
    out = _tc_head(h, batch, g_count, fcW1, fcb1, fcW2, fcb2)
    return out.reshape(-1)


# speedup vs baseline: 11.7748x; 1.1829x over previous
import functools

import jax
import jax.numpy as jnp
from jax import lax
from jax.experimental import pallas as pl
from jax.experimental.pallas import tpu as pltpu
from jax.experimental.pallas import tpu_sc as plsc

_NC = 2
_NS = 16
_L = 16


def _sc_agg(x, packed, ew):
    n, d = x.shape
    e = packed.shape[0]
    nw = _NC * _NS
    epw = e // nw
    K = 80
    nch = epw // K
    rpt = (n // (8 * _NS)) * 8
    rem = n - _NS * rpt
    assert e == nw * epw and epw == nch * K
    assert d % _L == 0 and K % 8 == 0 and epw % 8 == 0
    assert rem % 8 == 0 and rem <= K

    segs = []
    off = 0
    while off + _L <= K:
        segs.append((off, 0))
        off += _L
    if off < K:
        segs.append((K - _L, _L - (K - off)))

    mesh = plsc.VectorSubcoreMesh(core_axis_name="c", subcore_axis_name="s")

    @functools.partial(
        pl.kernel,
        out_type=jax.ShapeDtypeStruct((_NC, n, d), jnp.float32),
        mesh=mesh,
        scratch_types=[
            pltpu.VMEM((K,), jnp.int32),
            pltpu.VMEM((K,), jnp.int32),
            pltpu.VMEM((K,), jnp.int32),
            pltpu.VMEM((K,), jnp.int32),
            pltpu.VMEM((2, K), jnp.int32),
            pltpu.VMEM((2, K), jnp.int32),
            pltpu.VMEM((K,), jnp.float32),
            pltpu.VMEM((K,), jnp.float32),
            pltpu.VMEM((K, d), jnp.float32),
            pltpu.VMEM((K, d), jnp.float32),
            pltpu.VMEM((K, d), jnp.float32),
            pltpu.VMEM((K, d), jnp.float32),
            pltpu.VMEM_SHARED((n, d), jnp.float32),
            pltpu.SemaphoreType.DMA,
            pltpu.SemaphoreType.DMA,
            pltpu.SemaphoreType.DMA,
            pltpu.SemaphoreType.DMA,
            pltpu.SemaphoreType.DMA,
            pltpu.SemaphoreType.DMA,
            pltpu.SemaphoreType.DMA,
            pltpu.SemaphoreType.DMA,
        ],
    )
    def agg_kernel(x_hbm, packed_hbm, ew_hbm, out_hbm,
                   pkb_a, pkb_b, srcb_a, srcb_b, dstb_a, dstb_b, ewb_a, ewb_b,
                   ga_v, gb_v, sa_v, sb_v, acc_sh,
                   sem_pa, sem_pb, sem_ea, sem_eb,
                   sem_ga, sem_gb, sem_sa, sem_sb):
        c = lax.axis_index("c")
        s = lax.axis_index("s")
        wid = c * _NS + s
        row0 = s * rpt
        ch0 = wid * nch

        def pk_src(i):
            return packed_hbm.at[pl.ds((ch0 + i) * K, K)]

        def ew_src(i):
            return ew_hbm.at[pl.ds((ch0 + i) * K, K)]

        def pkload(i, pkb, sem):
            pltpu.async_copy(pk_src(i), pkb, sem)

        def wait_pk(i, pkb, sem):
            pltpu.make_async_copy(pk_src(i), pkb, sem).wait()

        def ewload(i, ewb, sem):
            pltpu.async_copy(ew_src(i), ewb, sem)

        def wait_ew(i, ewb, sem):
            pltpu.make_async_copy(ew_src(i), ewb, sem).wait()

        def unpack(i, pkb, srcb, dstb, slot):
            for soff, _l0 in segs:
                p = pkb[pl.ds(soff, _L)]
                srcb[pl.ds(soff, _L)] = p & jnp.int32(0xFFFF)
                dstb[slot, pl.ds(soff, _L)] = lax.shift_right_logical(
                    p, jnp.int32(16))

        def gather(srcb, g_v, sem):
            pltpu.async_copy(x_hbm.at[srcb], g_v, sem)

        def wait_gather(srcb, g_v, sem):
            pltpu.make_async_copy(x_hbm.at[srcb], g_v, sem).wait()

        def scatter(dstb, slot, s_v, sem):
            pltpu.async_copy(s_v, acc_sh.at[dstb.at[slot]], sem, add=True)

        def wait_scatter(dstb, slot, s_v, sem):
            pltpu.make_async_copy(s_v, acc_sh.at[dstb.at[slot]], sem).wait()

        pltpu.sync_copy(pk_src(0), pkb_a)
        pltpu.sync_copy(pk_src(1), pkb_b)
        pltpu.sync_copy(ew_src(0), ewb_a)
        pltpu.sync_copy(ew_src(1), ewb_b)
        unpack(0, pkb_a, srcb_a, dstb_a, 0)
        unpack(1, pkb_b, srcb_b, dstb_b, 0)
        gather(srcb_a, ga_v, sem_ga)
        gather(srcb_b, gb_v, sem_gb)
        if nch > 2:
            pkload(2, pkb_a, sem_pa)
        if nch > 3:
            pkload(3, pkb_b, sem_pb)

        z16 = jnp.zeros((_L,), jnp.float32)
        for k in range(K):
            for j in range(d // _L):
                sa_v[k, pl.ds(j * _L, _L)] = z16
        nfull = rpt // K
        tail = rpt - nfull * K
        for t in range(nfull):
            pltpu.sync_copy(sa_v, acc_sh.at[pl.ds(row0 + t * K, K)])
        if tail:
            pltpu.sync_copy(sa_v.at[pl.ds(0, tail)],
                            acc_sh.at[pl.ds(row0 + nfull * K, tail)])
        if rem:
            @pl.when(s == _NS - 1)
            def _():
                pltpu.sync_copy(sa_v.at[pl.ds(0, rem)],
                                acc_sh.at[pl.ds(n - rem, rem)])
        plsc.subcore_barrier()

        def scale(g_v, s_v, ewb):
            for soff, l0 in segs:
                ewv = ewb[pl.ds(soff, _L)]
                for l in range(l0, _L):
                    w = lax.gather(
                        ewv, jnp.full((_L, 1), l, jnp.int32),
                        lax.GatherDimensionNumbers(
                            offset_dims=(), collapsed_slice_dims=(0,),
                            start_index_map=(0,)),
                        (1,), mode=lax.GatherScatterMode.PROMISE_IN_BOUNDS)
                    k = soff + l
                    for j in range(d // _L):
                        s_v[k, pl.ds(j * _L, _L)] = (
                            g_v[k, pl.ds(j * _L, _L)] * w)

        def step(i, slot_cur, slot_next, pkb, srcb, dstb, ewb,
                 g_v, s_v, sem_p, sem_e, sem_g, sem_s):
            wait_gather(srcb, g_v, sem_g)

            @pl.when(i >= 2)
            def _():
                wait_scatter(dstb, slot_next, s_v, sem_s)
                wait_ew(i, ewb, sem_e)

            scale(g_v, s_v, ewb)
            scatter(dstb, slot_cur, s_v, sem_s)

            @pl.when(i + 2 < nch)
            def _():
                ewload(i + 2, ewb, sem_e)
                wait_pk(i + 2, pkb, sem_p)
                unpack(i + 2, pkb, srcb, dstb, slot_next)
                gather(srcb, g_v, sem_g)

            @pl.when(i + 4 < nch)
            def _():
                pkload(i + 4, pkb, sem_p)

        def step_a(i, slot_cur, slot_next):
            step(i, slot_cur, slot_next, pkb_a, srcb_a, dstb_a, ewb_a,
                 ga_v, sa_v, sem_pa, sem_ea, sem_ga, sem_sa)

        def step_b(i, slot_cur, slot_next):
            step(i, slot_cur, slot_next, pkb_b, srcb_b, dstb_b, ewb_b,
                 gb_v, sb_v, sem_pb, sem_eb, sem_gb, sem_sb)

        def pair(p, carry):
            slot_cur = lax.rem(p, 2)
            slot_next = 1 - slot_cur
            step_a(2 * p, slot_cur, slot_next)
            step_b(2 * p + 1, slot_cur, slot_next)
            return carry

        lax.fori_loop(0, nch // 2, pair, 0)
        if nch % 2:
            lslot = (nch // 2) % 2
            step_a(nch - 1, lslot, 1 - lslot)
            wait_scatter(dstb_a, lslot, sa_v, sem_sa)
            wait_scatter(dstb_b, 1 - lslot, sb_v, sem_sb)
        else:
            lslot = (nch // 2 - 1) % 2
            wait_scatter(dstb_a, lslot, sa_v, sem_sa)
            wait_scatter(dstb_b, lslot, sb_v, sem_sb)

        plsc.subcore_barrier()
        pltpu.sync_copy(acc_sh.at[pl.ds(row0, rpt)],
                        out_hbm.at[c].at[pl.ds(row0, rpt)])
        if rem:
            @pl.when(s == _NS - 1)
            def _():
                pltpu.sync_copy(acc_sh.at[pl.ds(n - rem, rem)],
                                out_hbm.at[c].at[pl.ds(n - rem, rem)])

    return agg_kernel(x, packed, ew)


def _layer_body(relu, h_ref, a_ref, eps_ref, w_ref, b_ref, g_ref, bb_ref, o_ref):
    agg = a_ref[0] + a_ref[1]
    eps = eps_ref[0]
    h = (1.0 + eps) * h_ref[...] + agg
    y = jnp.dot(h, w_ref[...], preferred_element_type=jnp.float32,
                precision=lax.Precision.HIGHEST) + b_ref[...]
    if relu:
        y = jnp.maximum(y, 0.0)
    m = jnp.mean(y, axis=0, keepdims=True)
    v = jnp.mean((y - m) * (y - m), axis=0, keepdims=True)
    o_ref[...] = (y - m) * lax.rsqrt(v + 1e-5) * g_ref[...] + bb_ref[...]


def _tc_layer(h, aggp, eps, w, b, g, bb, relu):
    n, dd = h.shape
    hh = w.shape[1]
    return pl.pallas_call(
        functools.partial(_layer_body, relu),
        out_shape=jax.ShapeDtypeStruct((n, hh), jnp.float32),
        in_specs=[
            pl.BlockSpec(memory_space=pltpu.VMEM),
            pl.BlockSpec(memory_space=pltpu.VMEM),
            pl.BlockSpec(memory_space=pltpu.SMEM),
            pl.BlockSpec(memory_space=pltpu.VMEM),
            pl.BlockSpec(memory_space=pltpu.VMEM),
            pl.BlockSpec(memory_space=pltpu.VMEM),
            pl.BlockSpec(memory_space=pltpu.VMEM),
        ],
        out_specs=pl.BlockSpec(memory_space=pltpu.VMEM),
    )(h, aggp, eps.reshape(1), w, b.reshape(1, -1), g.reshape(1, -1),
      bb.reshape(1, -1))


def _head_body(g_count, h_ref, batch_ref, w1_ref, b1_ref, w2_ref, b2_ref, o_ref):
    h = h_ref[...]
    n = h.shape[0]
    bt = batch_ref[...]
    gid = lax.broadcasted_iota(jnp.int32, (g_count, n), 0)
    oh = (gid == bt).astype(jnp.float32)
    pooled = jnp.dot(oh, h, preferred_element_type=jnp.float32,
                     precision=lax.Precision.HIGHEST)
    pooled = jnp.maximum(pooled, 0.0)
    y = jnp.maximum(jnp.dot(pooled, w1_ref[...], preferred_element_type=jnp.float32,
                            precision=lax.Precision.HIGHEST) + b1_ref[...], 0.0)
    o_ref[...] = jnp.dot(y, w2_ref[...], preferred_element_type=jnp.float32,
                         precision=lax.Precision.HIGHEST) + b2_ref[...]


def _tc_head(h, batch, g_count, w1, b1, w2, b2):
    return pl.pallas_call(
        functools.partial(_head_body, g_count),
        out_shape=jax.ShapeDtypeStruct((g_count, 1), jnp.float32),
    )(h, batch.reshape(1, -1), w1, b1.reshape(1, -1), w2, b2.reshape(1, -1))


def kernel(x, edge_index, edge_weight, batch, eps1, W1, b1, eps2, W2, b2,
           eps3, W3, b3, g1, bb1, g2, bb2, g3, bb3, fcW1, fcb1, fcW2, fcb2):
    src = edge_index[0]
    dst = edge_index[1]
    packed = jnp.bitwise_or(lax.shift_left(dst, 16), src)
    g_count = 64

    aggp = _sc_agg(x, packed, edge_weight)
    h = _tc_layer(x, aggp, eps1, W1, b1, g1, bb1, relu=True)
    aggp = _sc_agg(h, packed, edge_weight)
    h = _tc_layer(h, aggp, eps2, W2, b2, g2, bb2, relu=True)
    aggp = _sc_agg(h, packed, edge_weight)
    h = _tc_layer(h, aggp, eps3, W3, b3, g3, bb3, relu=False)
    out = _tc_head(h, batch, g_count, fcW1, fcb1, fcW2, fcb2)
    return out.reshape(-1)


# speedup vs baseline: 12.0164x; 1.0205x over previous
import functools

import jax
import jax.numpy as jnp
from jax import lax
from jax.experimental import pallas as pl
from jax.experimental.pallas import tpu as pltpu
from jax.experimental.pallas import tpu_sc as plsc

_NC = 2
_NS = 16
_L = 16


def _sc_agg(x, packed, ew):
    n, d = x.shape
    e = packed.shape[0]
    nw = _NC * _NS
    epw = e // nw
    K = 80
    nch = epw // K
    rpt = (n // (8 * _NS)) * 8
    rem = n - _NS * rpt
    assert e == nw * epw and epw == nch * K
    assert d % _L == 0 and K % 8 == 0 and epw % 8 == 0
    assert rem % 8 == 0 and rem <= K

    segs = []
    off = 0
    while off + _L <= K:
        segs.append((off, 0))
        off += _L
    if off < K:
        segs.append((K - _L, _L - (K - off)))

    mesh = plsc.VectorSubcoreMesh(core_axis_name="c", subcore_axis_name="s")

    @functools.partial(
        pl.kernel,
        out_type=jax.ShapeDtypeStruct((_NC, n, d), jnp.float32),
        mesh=mesh,
        scratch_types=[
            pltpu.VMEM((K,), jnp.int32),
            pltpu.VMEM((K,), jnp.int32),
            pltpu.VMEM((K,), jnp.int32),
            pltpu.VMEM((K,), jnp.int32),
            pltpu.VMEM((2, K), jnp.int32),
            pltpu.VMEM((2, K), jnp.int32),
            pltpu.VMEM((K,), jnp.float32),
            pltpu.VMEM((K,), jnp.float32),
            pltpu.VMEM((K, d), jnp.float32),
            pltpu.VMEM((K, d), jnp.float32),
            pltpu.VMEM((K, d), jnp.float32),
            pltpu.VMEM((K, d), jnp.float32),
            pltpu.VMEM_SHARED((n, d), jnp.float32),
            pltpu.SemaphoreType.DMA,
            pltpu.SemaphoreType.DMA,
            pltpu.SemaphoreType.DMA,
            pltpu.SemaphoreType.DMA,
            pltpu.SemaphoreType.DMA,
            pltpu.SemaphoreType.DMA,
            pltpu.SemaphoreType.DMA,
            pltpu.SemaphoreType.DMA,
        ],
    )
    def agg_kernel(x_hbm, packed_hbm, ew_hbm, out_hbm,
                   pkb_a, pkb_b, srcb_a, srcb_b, dstb_a, dstb_b, ewb_a, ewb_b,
                   ga_v, gb_v, sa_v, sb_v, acc_sh,
                   sem_pa, sem_pb, sem_ea, sem_eb,
                   sem_ga, sem_gb, sem_sa, sem_sb):
        c = lax.axis_index("c")
        s = lax.axis_index("s")
        wid = c * _NS + s
        row0 = s * rpt
        ch0 = wid * nch

        def pk_src(i):
            return packed_hbm.at[pl.ds((ch0 + i) * K, K)]

        def ew_src(i):
            return ew_hbm.at[pl.ds((ch0 + i) * K, K)]

        def pkload(i, pkb, sem):
            pltpu.async_copy(pk_src(i), pkb, sem)

        def wait_pk(i, pkb, sem):
            pltpu.make_async_copy(pk_src(i), pkb, sem).wait()

        def ewload(i, ewb, sem):
            pltpu.async_copy(ew_src(i), ewb, sem)

        def wait_ew(i, ewb, sem):
            pltpu.make_async_copy(ew_src(i), ewb, sem).wait()

        def unpack(i, pkb, srcb, dstb, slot):
            for soff, _l0 in segs:
                p = pkb[pl.ds(soff, _L)]
                srcb[pl.ds(soff, _L)] = p & jnp.int32(0xFFFF)
                dstb[slot, pl.ds(soff, _L)] = lax.shift_right_logical(
                    p, jnp.int32(16))

        def gather(srcb, g_v, sem):
            pltpu.async_copy(x_hbm.at[srcb], g_v, sem)

        def wait_gather(srcb, g_v, sem):
            pltpu.make_async_copy(x_hbm.at[srcb], g_v, sem).wait()

        def scatter(dstb, slot, s_v, sem):
            pltpu.async_copy(s_v, acc_sh.at[dstb.at[slot]], sem, add=True)

        def wait_scatter(dstb, slot, s_v, sem):
            pltpu.make_async_copy(s_v, acc_sh.at[dstb.at[slot]], sem).wait()

        pltpu.sync_copy(pk_src(0), pkb_a)
        pltpu.sync_copy(pk_src(1), pkb_b)
        pltpu.sync_copy(ew_src(0), ewb_a)
        pltpu.sync_copy(ew_src(1), ewb_b)
        unpack(0, pkb_a, srcb_a, dstb_a, 0)
        unpack(1, pkb_b, srcb_b, dstb_b, 0)
        gather(srcb_a, ga_v, sem_ga)
        gather(srcb_b, gb_v, sem_gb)
        if nch > 2:
            pkload(2, pkb_a, sem_pa)
        if nch > 3:
            pkload(3, pkb_b, sem_pb)

        z16 = jnp.zeros((_L,), jnp.float32)
        for k in range(K):
            for j in range(d // _L):
                sa_v[k, pl.ds(j * _L, _L)] = z16
        nfull = rpt // K
        tail = rpt - nfull * K
        for t in range(nfull):
            pltpu.sync_copy(sa_v, acc_sh.at[pl.ds(row0 + t * K, K)])
        if tail:
            pltpu.sync_copy(sa_v.at[pl.ds(0, tail)],
                            acc_sh.at[pl.ds(row0 + nfull * K, tail)])
        if rem:
            @pl.when(s == _NS - 1)
            def _():
                pltpu.sync_copy(sa_v.at[pl.ds(0, rem)],
                                acc_sh.at[pl.ds(n - rem, rem)])
        plsc.subcore_barrier()

        def scale(g_v, s_v, ewb):
            for soff, l0 in segs:
                ewv = ewb[pl.ds(soff, _L)]
                for l in range(l0, _L):
                    w = lax.gather(
                        ewv, jnp.full((_L, 1), l, jnp.int32),
                        lax.GatherDimensionNumbers(
                            offset_dims=(), collapsed_slice_dims=(0,),
                            start_index_map=(0,)),
                        (1,), mode=lax.GatherScatterMode.PROMISE_IN_BOUNDS)
                    k = soff + l
                    for j in range(d // _L):
                        s_v[k, pl.ds(j * _L, _L)] = (
                            g_v[k, pl.ds(j * _L, _L)] * w)

        def step(i, slot_cur, slot_next, pkb, srcb, dstb, ewb,
                 g_v, s_v, sem_p, sem_e, sem_g, sem_s):
            wait_gather(srcb, g_v, sem_g)

            @pl.when(i >= 2)
            def _():
                wait_scatter(dstb, slot_next, s_v, sem_s)
                wait_ew(i, ewb, sem_e)

            scale(g_v, s_v, ewb)
            scatter(dstb, slot_cur, s_v, sem_s)

            @pl.when(i + 2 < nch)
            def _():
                ewload(i + 2, ewb, sem_e)
                wait_pk(i + 2, pkb, sem_p)
                unpack(i + 2, pkb, srcb, dstb, slot_next)
                gather(srcb, g_v, sem_g)

            @pl.when(i + 4 < nch)
            def _():
                pkload(i + 4, pkb, sem_p)

        def step_a(i, slot_cur, slot_next):
            step(i, slot_cur, slot_next, pkb_a, srcb_a, dstb_a, ewb_a,
                 ga_v, sa_v, sem_pa, sem_ea, sem_ga, sem_sa)

        def step_b(i, slot_cur, slot_next):
            step(i, slot_cur, slot_next, pkb_b, srcb_b, dstb_b, ewb_b,
                 gb_v, sb_v, sem_pb, sem_eb, sem_gb, sem_sb)

        def pair(p, carry):
            slot_cur = lax.rem(p, 2)
            slot_next = 1 - slot_cur
            step_a(2 * p, slot_cur, slot_next)
            step_b(2 * p + 1, slot_cur, slot_next)
            return carry

        lax.fori_loop(0, nch // 2, pair, 0)
        if nch % 2:
            lslot = (nch // 2) % 2
            step_a(nch - 1, lslot, 1 - lslot)
            wait_scatter(dstb_a, lslot, sa_v, sem_sa)
            wait_scatter(dstb_b, 1 - lslot, sb_v, sem_sb)
        else:
            lslot = (nch // 2 - 1) % 2
            wait_scatter(dstb_a, lslot, sa_v, sem_sa)
            wait_scatter(dstb_b, lslot, sb_v, sem_sb)

        plsc.subcore_barrier()
        pltpu.sync_copy(acc_sh.at[pl.ds(row0, rpt)],
                        out_hbm.at[c].at[pl.ds(row0, rpt)])
        if rem:
            @pl.when(s == _NS - 1)
            def _():
                pltpu.sync_copy(acc_sh.at[pl.ds(n - rem, rem)],
                                out_hbm.at[c].at[pl.ds(n - rem, rem)])

    return agg_kernel(x, packed, ew)


def _layer_body(relu, h_ref, a_ref, eps_ref, w_ref, b_ref, g_ref, bb_ref, o_ref):
    agg = a_ref[0] + a_ref[1]
    eps = eps_ref[0]
    h = (1.0 + eps) * h_ref[...] + agg
    y = jnp.dot(h, w_ref[...], preferred_element_type=jnp.float32) + b_ref[...]
    if relu:
        y = jnp.maximum(y, 0.0)
    m = jnp.mean(y, axis=0, keepdims=True)
    v = jnp.mean((y - m) * (y - m), axis=0, keepdims=True)
    o_ref[...] = (y - m) * lax.rsqrt(v + 1e-5) * g_ref[...] + bb_ref[...]


def _tc_layer(h, aggp, eps, w, b, g, bb, relu):
    n, dd = h.shape
    hh = w.shape[1]
    return pl.pallas_call(
        functools.partial(_layer_body, relu),
        out_shape=jax.ShapeDtypeStruct((n, hh), jnp.float32),
        in_specs=[
            pl.BlockSpec(memory_space=pltpu.VMEM),
            pl.BlockSpec(memory_space=pltpu.VMEM),
            pl.BlockSpec(memory_space=pltpu.SMEM),
            pl.BlockSpec(memory_space=pltpu.VMEM),
            pl.BlockSpec(memory_space=pltpu.VMEM),
            pl.BlockSpec(memory_space=pltpu.VMEM),
            pl.BlockSpec(memory_space=pltpu.VMEM),
        ],
        out_specs=pl.BlockSpec(memory_space=pltpu.VMEM),
    )(h, aggp, eps.reshape(1), w, b.reshape(1, -1), g.reshape(1, -1),
      bb.reshape(1, -1))


def _head_body(g_count, h_ref, batch_ref, w1_ref, b1_ref, w2_ref, b2_ref, o_ref):
    h = h_ref[...]
    n = h.shape[0]
    bt = batch_ref[...]
    gid = lax.broadcasted_iota(jnp.int32, (g_count, n), 0)
    oh = (gid == bt).astype(jnp.float32)
    pooled = jnp.dot(oh, h, preferred_element_type=jnp.float32,
                     precision=lax.Precision.HIGHEST)
    pooled = jnp.maximum(pooled, 0.0)
    y = jnp.maximum(jnp.dot(pooled, w1_ref[...],
                            preferred_element_type=jnp.float32)
                    + b1_ref[...], 0.0)
    o_ref[...] = jnp.dot(y, w2_ref[...],
                         preferred_element_type=jnp.float32) + b2_ref[...]


def _tc_head(h, batch, g_count, w1, b1, w2, b2):
    return pl.pallas_call(
        functools.partial(_head_body, g_count),
        out_shape=jax.ShapeDtypeStruct((g_count, 1), jnp.float32),
    )(h, batch.reshape(1, -1), w1, b1.reshape(1, -1), w2, b2.reshape(1, -1))


def kernel(x, edge_index, edge_weight, batch, eps1, W1, b1, eps2, W2, b2,
           eps3, W3, b3, g1, bb1, g2, bb2, g3, bb3, fcW1, fcb1, fcW2, fcb2):
    src = edge_index[0]
    dst = edge_index[1]
    packed = jnp.bitwise_or(lax.shift_left(dst, 16), src)
    g_count = 64

    aggp = _sc_agg(x, packed, edge_weight)
    h = _tc_layer(x, aggp, eps1, W1, b1, g1, bb1, relu=True)
    aggp = _sc_agg(h, packed, edge_weight)
    h = _tc_layer(h, aggp, eps2, W2, b2, g2, bb2, relu=True)
    aggp = _sc_agg(h, packed, edge_weight)
    h = _tc_layer(h, aggp, eps3, W3, b3, g3, bb3, relu=False)
    out = _tc_head(h, batch, g_count, fcW1, fcb1, fcW2, fcb2)
    return out.reshape(-1)
